# baseline (speedup 1.0000x reference)
import jax
import jax.numpy as jnp
from jax import lax
from jax.experimental import pallas as pl
from jax.experimental.pallas import tpu as pltpu

_LN_EPS = 1e-5
_NUM_ENC = 8


def _fused_body(cb_ref, cbt_ref, w_ref, b_ref, gz_ref, bz_ref, gm_ref, bm_ref,
                m_ref, z_ref, mo_ref, zo_ref):
    row_tile, n_res, z_dim = zo_ref.shape

    a = cb_ref[...]
    bt = cbt_ref[...]
    dc = a[:, 0:1] - bt[0:1, :]
    d2 = dc * dc
    for c in (1, 2):
        dc = a[:, c:c + 1] - bt[c:c + 1, :]
        d2 = d2 + dc * dc
    dist = jnp.sqrt(d2)

    s = jnp.sin(dist * jnp.float32(2.0 ** (1 - _NUM_ENC)))
    c = jnp.cos(dist * jnp.float32(2.0 ** (1 - _NUM_ENC)))
    sin_p = [None] * _NUM_ENC
    cos_p = [None] * _NUM_ENC
    sin_p[_NUM_ENC - 1], cos_p[_NUM_ENC - 1] = s, c
    for k in range(_NUM_ENC - 2, -1, -1):
        s, c = 2.0 * s * c, 1.0 - 2.0 * s * s
        sin_p[k], cos_p[k] = s, c

    zb = z_ref[...]
    mu = jnp.mean(zb, axis=-1, keepdims=True)
    var = jnp.mean(zb * zb, axis=-1, keepdims=True) - mu * mu
    zn = (zb - mu) * lax.rsqrt(var + _LN_EPS) * gz_ref[...].reshape(1, 1, z_dim) \
        + (bz_ref[...] + b_ref[...]).reshape(1, 1, z_dim)

    wmat = w_ref[...].astype(jnp.bfloat16)
    projs = []
    for i in range(row_tile):
        feats_t = jnp.concatenate(
            [p[i:i + 1, :] for p in sin_p]
            + [p[i:i + 1, :] for p in cos_p]
            + [dist[i:i + 1, :]], axis=0)
        projs.append(lax.dot_general(
            feats_t.astype(jnp.bfloat16), wmat, (((0,), (0,)), ((), ())),
            preferred_element_type=jnp.float32)[None])
    zo_ref[...] = zn + jnp.concatenate(projs, axis=0)

    mb = m_ref[...]
    mmu = jnp.mean(mb, axis=-1, keepdims=True)
    mvar = jnp.mean((mb - mmu) ** 2, axis=-1, keepdims=True)
    mo_ref[...] = (mb - mmu) * lax.rsqrt(mvar + _LN_EPS) * gm_ref[...] + bm_ref[...]


def kernel(m, z, x, w, b, gz, bz, gm, bm):
    n_res, _, z_dim = z.shape
    n_seq, _, m_dim = m.shape
    cb = x[:, -1]
    cbt = jnp.transpose(cb)
    m2 = m.reshape(n_seq * n_res, m_dim)

    row_tile = 16
    grid = (n_res // row_tile,)
    m_tile = (n_seq * n_res) // grid[0]

    bcast = lambda i: (0, 0)
    mo, zo = pl.pallas_call(
        _fused_body,
        out_shape=(
            jax.ShapeDtypeStruct((n_seq * n_res, m_dim), m.dtype),
            jax.ShapeDtypeStruct((n_res, n_res, z_dim), z.dtype),
        ),
        grid=grid,
        in_specs=[
            pl.BlockSpec((row_tile, 3), lambda i: (i, 0)),
            pl.BlockSpec((3, n_res), bcast),
            pl.BlockSpec((2 * _NUM_ENC + 1, z_dim), bcast),
            pl.BlockSpec((1, z_dim), bcast),
            pl.BlockSpec((1, z_dim), bcast),
            pl.BlockSpec((1, z_dim), bcast),
            pl.BlockSpec((1, m_dim), bcast),
            pl.BlockSpec((1, m_dim), bcast),
            pl.BlockSpec((m_tile, m_dim), lambda i: (i, 0)),
            pl.BlockSpec((row_tile, n_res, z_dim), lambda i: (i, 0, 0)),
        ],
        out_specs=(
            pl.BlockSpec((m_tile, m_dim), lambda i: (i, 0)),
            pl.BlockSpec((row_tile, n_res, z_dim), lambda i: (i, 0, 0)),
        ),
        compiler_params=pltpu.CompilerParams(
            dimension_semantics=("arbitrary",),
            vmem_limit_bytes=64 * 1024 * 1024,
        ),
    )(cb, cbt, w, b, gz, bz, gm, bm, m2, z)

    return mo.reshape(n_seq, n_res, m_dim), zo


# speedup vs baseline: 1.0431x; 1.0431x over previous
import jax
import jax.numpy as jnp
from jax import lax
from jax.experimental import pallas as pl
from jax.experimental.pallas import tpu as pltpu

_LN_EPS = 1e-5
_NUM_ENC = 8


def _fused_body(cb_ref, cbt_ref, w_ref, b_ref, gz_ref, bz_ref, gm_ref, bm_ref,
                m_ref, z_ref, mo_ref, zo_ref):
    row_tile, n_res, z_dim = zo_ref.shape

    a = cb_ref[...]
    bt = cbt_ref[...]
    dc = a[:, 0:1] - bt[0:1, :]
    d2 = dc * dc
    for c in (1, 2):
        dc = a[:, c:c + 1] - bt[c:c + 1, :]
        d2 = d2 + dc * dc
    dist = jnp.sqrt(d2)

    s = jnp.sin(dist * jnp.float32(2.0 ** (1 - _NUM_ENC)))
    c = jnp.cos(dist * jnp.float32(2.0 ** (1 - _NUM_ENC)))
    sin_p = [None] * _NUM_ENC
    cos_p = [None] * _NUM_ENC
    sin_p[_NUM_ENC - 1], cos_p[_NUM_ENC - 1] = s, c
    for k in range(_NUM_ENC - 2, -1, -1):
        s, c = 2.0 * s * c, 1.0 - 2.0 * s * s
        sin_p[k], cos_p[k] = s, c

    zb = z_ref[...]
    mu = jnp.mean(zb, axis=-1, keepdims=True)
    var = jnp.mean((zb - mu) ** 2, axis=-1, keepdims=True)
    zn = (zb - mu) * lax.rsqrt(var + _LN_EPS) * gz_ref[...].reshape(1, 1, z_dim) \
        + (bz_ref[...] + b_ref[...]).reshape(1, 1, z_dim)

    wmat = w_ref[...].astype(jnp.bfloat16)
    projs = []
    for i in range(row_tile):
        feats_t = jnp.concatenate(
            [p[i:i + 1, :] for p in sin_p]
            + [p[i:i + 1, :] for p in cos_p]
            + [dist[i:i + 1, :]], axis=0)
        projs.append(lax.dot_general(
            feats_t.astype(jnp.bfloat16), wmat, (((0,), (0,)), ((), ())),
            preferred_element_type=jnp.float32)[None])
    zo_ref[...] = zn + jnp.concatenate(projs, axis=0)

    mb = m_ref[...]
    mmu = jnp.mean(mb, axis=-1, keepdims=True)
    mvar = jnp.mean((mb - mmu) ** 2, axis=-1, keepdims=True)
    mo_ref[...] = (mb - mmu) * lax.rsqrt(mvar + _LN_EPS) * gm_ref[...] + bm_ref[...]


def kernel(m, z, x, w, b, gz, bz, gm, bm):
    n_res, _, z_dim = z.shape
    n_seq, _, m_dim = m.shape
    cb = x[:, -1]
    cbt = jnp.transpose(cb)
    m2 = m.reshape(n_seq * n_res, m_dim)

    row_tile = 16
    grid = (n_res // row_tile,)
    m_tile = (n_seq * n_res) // grid[0]

    bcast = lambda i: (0, 0)
    mo, zo = pl.pallas_call(
        _fused_body,
        out_shape=(
            jax.ShapeDtypeStruct((n_seq * n_res, m_dim), m.dtype),
            jax.ShapeDtypeStruct((n_res, n_res, z_dim), z.dtype),
        ),
        grid=grid,
        in_specs=[
            pl.BlockSpec((row_tile, 3), lambda i: (i, 0)),
            pl.BlockSpec((3, n_res), bcast),
            pl.BlockSpec((2 * _NUM_ENC + 1, z_dim), bcast),
            pl.BlockSpec((1, z_dim), bcast),
            pl.BlockSpec((1, z_dim), bcast),
            pl.BlockSpec((1, z_dim), bcast),
            pl.BlockSpec((1, m_dim), bcast),
            pl.BlockSpec((1, m_dim), bcast),
            pl.BlockSpec((m_tile, m_dim), lambda i: (i, 0)),
            pl.BlockSpec((row_tile, n_res, z_dim), lambda i: (i, 0, 0)),
        ],
        out_specs=(
            pl.BlockSpec((m_tile, m_dim), lambda i: (i, 0)),
            pl.BlockSpec((row_tile, n_res, z_dim), lambda i: (i, 0, 0)),
        ),
        compiler_params=pltpu.CompilerParams(
            dimension_semantics=("arbitrary",),
            vmem_limit_bytes=64 * 1024 * 1024,
        ),
    )(cb, cbt, w, b, gz, bz, gm, bm, m2, z)

    return mo.reshape(n_seq, n_res, m_dim), zo
